# baseline (speedup 1.0000x reference)
import functools

import jax
import jax.numpy as jnp
from jax import lax
from jax.experimental import pallas as pl
from jax.experimental.pallas import tpu as pltpu
from jax.experimental.pallas import tpu_sc as plsc

_RESIZE = 224.0
_PATCH = 14.0
_GRID = 16

_NC = 2
_NS = 16
_NW = _NC * _NS


def _prep_body(params_ref, scale_ref, x_ref, y_ref, z_ref, pt_ref, w1b_ref,
               idx_ref, g_ref):
    b = pl.program_id(0)
    x = x_ref[0]
    y = y_ref[0]
    z = z_ref[0]
    rt = [params_ref[0, 0, j] for j in range(12)]
    kk = [params_ref[0, 0, 12 + j] for j in range(9)]
    cam = [rt[4 * i + 0] * x + rt[4 * i + 1] * y + rt[4 * i + 2] * z +
           rt[4 * i + 3] for i in range(3)]
    cam = [c.astype(jnp.bfloat16).astype(jnp.float32) for c in cam]
    pix = [kk[3 * i + 0] * cam[0] + kk[3 * i + 1] * cam[1] +
           kk[3 * i + 2] * cam[2] for i in range(3)]
    den = pix[2] + 1e-6
    u = (pix[0] / den) * scale_ref[0, 0]
    v = (pix[1] / den) * scale_ref[0, 1]
    px = jnp.clip((u / _PATCH).astype(jnp.int32), 0, _GRID - 1)
    py = jnp.clip((v / _PATCH).astype(jnp.int32), 0, _GRID - 1)
    m = pt_ref.shape[2]
    idx_ref[0] = px * _GRID + py + b * m

    t = (pt_ref[0, 0] + pt_ref[0, 1]) * 0.5
    g_ref[0] = jnp.dot(t, w1b_ref[...], preferred_element_type=jnp.float32)


def _prep(params, scale, xs, ys, zs, patch_tokens, w1b):
    B, nv, M, dim = patch_tokens.shape
    V = xs.shape[2]
    hidden = w1b.shape[1]
    return pl.pallas_call(
        _prep_body,
        grid=(B,),
        in_specs=[
            pl.BlockSpec((1, 1, params.shape[2]), lambda b: (b, 0, 0),
                         memory_space=pltpu.SMEM),
            pl.BlockSpec((1, 2), lambda b: (0, 0), memory_space=pltpu.SMEM),
            pl.BlockSpec((1, 1, V), lambda b: (b, 0, 0)),
            pl.BlockSpec((1, 1, V), lambda b: (b, 0, 0)),
            pl.BlockSpec((1, 1, V), lambda b: (b, 0, 0)),
            pl.BlockSpec((1, nv, M, dim), lambda b: (b, 0, 0, 0)),
            pl.BlockSpec((dim, hidden), lambda b: (0, 0)),
        ],
        out_specs=[
            pl.BlockSpec((1, 1, V), lambda b: (b, 0, 0)),
            pl.BlockSpec((1, M, hidden), lambda b: (b, 0, 0)),
        ],
        out_shape=[
            jax.ShapeDtypeStruct((B, 1, V), jnp.int32),
            jax.ShapeDtypeStruct((B, M, hidden), jnp.float32),
        ],
    )(params, scale, xs, ys, zs, patch_tokens, w1b)


def _make_sc_gather(n_rows, hidden, chunk, nbuf):
    rows_per_w = n_rows // _NW
    n_chunks = rows_per_w // chunk
    n_super = n_chunks // nbuf
    assert nbuf == 2 and n_super * nbuf == n_chunks
    mesh = plsc.VectorSubcoreMesh(core_axis_name="c", subcore_axis_name="s")

    @functools.partial(
        pl.kernel,
        mesh=mesh,
        out_type=jax.ShapeDtypeStruct((n_rows, hidden), jnp.float32),
        scratch_types=[
            pltpu.VMEM((n_chunks, chunk), jnp.int32),
            pltpu.VMEM((nbuf, chunk, hidden), jnp.float32),
            pltpu.SemaphoreType.DMA,
            pltpu.SemaphoreType.DMA,
            pltpu.SemaphoreType.DMA,
            pltpu.SemaphoreType.DMA,
        ],
    )
    def gather_k(idx_hbm, g_hbm, out_hbm, idx_v, rows_v,
                 gsem0, gsem1, osem0, osem1):
        wid = lax.axis_index("s") * _NC + lax.axis_index("c")
        base = wid * rows_per_w
        gsems = [gsem0, gsem1]
        osems = [osem0, osem1]
        pltpu.sync_copy(idx_hbm.at[wid], idx_v)

        for b in range(nbuf):
            pltpu.async_copy(g_hbm.at[idx_v.at[b]], rows_v.at[b], gsems[b])

        @pl.loop(0, n_super)
        def _super(g):
            for b in range(nbuf):
                c = g * nbuf + b
                pltpu.make_async_copy(
                    g_hbm.at[pl.ds(0, chunk)], rows_v.at[b], gsems[b]).wait()
                pltpu.async_copy(
                    rows_v.at[b],
                    out_hbm.at[pl.ds(base + c * chunk, chunk)], osems[b])
                pltpu.make_async_copy(
                    rows_v.at[b], out_hbm.at[pl.ds(0, chunk)],
                    osems[b]).wait()

                @pl.when(g < n_super - 1)
                def _():
                    pltpu.async_copy(g_hbm.at[idx_v.at[c + nbuf]],
                                     rows_v.at[b], gsems[b])

    return gather_k


def _mlp_body(vf_ref, img_ref, w1a_ref, b1_ref, w2_ref, b2_ref, out_ref):
    h = jnp.dot(vf_ref[...], w1a_ref[...], preferred_element_type=jnp.float32)
    h = jnp.maximum(h + img_ref[...] + b1_ref[...], 0.0)
    out_ref[...] = (jnp.dot(h, w2_ref[...], preferred_element_type=jnp.float32)
                    + b2_ref[...])


def _mlp(vf, img, w1a, b1, w2, b2, tile):
    n, pfd = vf.shape
    hidden = w1a.shape[1]
    out_dim = w2.shape[1]
    return pl.pallas_call(
        _mlp_body,
        grid=(n // tile,),
        in_specs=[
            pl.BlockSpec((tile, pfd), lambda i: (i, 0)),
            pl.BlockSpec((tile, hidden), lambda i: (i, 0)),
            pl.BlockSpec((pfd, hidden), lambda i: (0, 0)),
            pl.BlockSpec((1, hidden), lambda i: (0, 0)),
            pl.BlockSpec((hidden, out_dim), lambda i: (0, 0)),
            pl.BlockSpec((1, out_dim), lambda i: (0, 0)),
        ],
        out_specs=pl.BlockSpec((tile, out_dim), lambda i: (i, 0)),
        out_shape=jax.ShapeDtypeStruct((n, out_dim), jnp.float32),
    )(vf, img, w1a, b1, w2, b2)


def kernel(patch_tokens, voxel_features, voxel_coords, image_sizes, K, Rt,
           W1, b1, W2, b2):
    B, nv, M, dim = patch_tokens.shape
    V = voxel_features.shape[1]
    pfd = voxel_features.shape[2]
    hidden = W1.shape[1]
    out_dim = W2.shape[1]
    n_rows = B * V

    def _bf(a):
        return a.astype(jnp.bfloat16).astype(jnp.float32)

    xs = _bf(voxel_coords[..., 0][:, None, :])
    ys = _bf(voxel_coords[..., 1][:, None, :])
    zs = _bf(voxel_coords[..., 2][:, None, :])
    params = _bf(jnp.concatenate([Rt.reshape(B, 1, 12), K.reshape(B, 1, 9)],
                                 axis=2))
    scale = (_RESIZE / image_sizes[0].astype(jnp.float32)).reshape(1, 2)

    idx3, g = _prep(params, scale, xs, ys, zs, patch_tokens, W1[pfd:, :])

    chunk, nbuf = 128, 2
    n_chunks = n_rows // _NW // chunk
    gather_k = _make_sc_gather(n_rows, hidden, chunk=chunk, nbuf=nbuf)
    img = gather_k(idx3.reshape(_NW, n_chunks, chunk), g.reshape(B * M, hidden))

    out = _mlp(voxel_features.reshape(n_rows, pfd), img, W1[:pfd, :],
               b1.reshape(1, hidden), W2, b2.reshape(1, out_dim), tile=2048)
    return out.reshape(B, V, out_dim)


# speedup vs baseline: 1.0087x; 1.0087x over previous
import functools

import jax
import jax.numpy as jnp
from jax import lax
from jax.experimental import pallas as pl
from jax.experimental.pallas import tpu as pltpu
from jax.experimental.pallas import tpu_sc as plsc

_RESIZE = 224.0
_PATCH = 14.0
_GRID = 16

_NC = 2
_NS = 16
_NW = _NC * _NS


def _prep_body(params_ref, scale_ref, x_ref, y_ref, z_ref, pt_ref, w1b_ref,
               idx_ref, g_ref):
    b = pl.program_id(0)
    x = x_ref[0]
    y = y_ref[0]
    z = z_ref[0]
    rt = [params_ref[0, 0, j] for j in range(12)]
    kk = [params_ref[0, 0, 12 + j] for j in range(9)]
    cam = [rt[4 * i + 0] * x + rt[4 * i + 1] * y + rt[4 * i + 2] * z +
           rt[4 * i + 3] for i in range(3)]
    cam = [c.astype(jnp.bfloat16).astype(jnp.float32) for c in cam]
    pix = [kk[3 * i + 0] * cam[0] + kk[3 * i + 1] * cam[1] +
           kk[3 * i + 2] * cam[2] for i in range(3)]
    den = pix[2] + 1e-6
    u = (pix[0] / den) * scale_ref[0, 0]
    v = (pix[1] / den) * scale_ref[0, 1]
    px = jnp.clip((u / _PATCH).astype(jnp.int32), 0, _GRID - 1)
    py = jnp.clip((v / _PATCH).astype(jnp.int32), 0, _GRID - 1)
    m = pt_ref.shape[2]
    idx_ref[0] = px * _GRID + py + b * m

    t = (pt_ref[0, 0] + pt_ref[0, 1]) * 0.5
    g_ref[0] = jnp.dot(t, w1b_ref[...], preferred_element_type=jnp.float32)


def _prep(params, scale, xs, ys, zs, patch_tokens, w1b):
    B, nv, M, dim = patch_tokens.shape
    V = xs.shape[2]
    hidden = w1b.shape[1]
    return pl.pallas_call(
        _prep_body,
        grid=(B,),
        in_specs=[
            pl.BlockSpec((1, 1, params.shape[2]), lambda b: (b, 0, 0),
                         memory_space=pltpu.SMEM),
            pl.BlockSpec((1, 2), lambda b: (0, 0), memory_space=pltpu.SMEM),
            pl.BlockSpec((1, 1, V), lambda b: (b, 0, 0)),
            pl.BlockSpec((1, 1, V), lambda b: (b, 0, 0)),
            pl.BlockSpec((1, 1, V), lambda b: (b, 0, 0)),
            pl.BlockSpec((1, nv, M, dim), lambda b: (b, 0, 0, 0)),
            pl.BlockSpec((dim, hidden), lambda b: (0, 0)),
        ],
        out_specs=[
            pl.BlockSpec((1, 1, V), lambda b: (b, 0, 0)),
            pl.BlockSpec((1, M, hidden), lambda b: (b, 0, 0)),
        ],
        out_shape=[
            jax.ShapeDtypeStruct((B, 1, V), jnp.int32),
            jax.ShapeDtypeStruct((B, M, hidden), jnp.float32),
        ],
    )(params, scale, xs, ys, zs, patch_tokens, w1b)


def _make_sc_gather(n_rows, n_table_rows, hidden, chunk, nbuf):
    rows_per_w = n_rows // _NW
    n_chunks = rows_per_w // chunk
    n_super = n_chunks // nbuf
    assert n_super * nbuf == n_chunks
    mesh = plsc.VectorSubcoreMesh(core_axis_name="c", subcore_axis_name="s")

    @functools.partial(
        pl.kernel,
        mesh=mesh,
        out_type=jax.ShapeDtypeStruct((n_rows, hidden), jnp.float32),
        scratch_types=[
            pltpu.VMEM((n_chunks, chunk), jnp.int32),
            pltpu.VMEM((nbuf, chunk, hidden), jnp.float32),
        ] + [pltpu.SemaphoreType.DMA] * (2 * nbuf),
    )
    def gather_k(idx_hbm, g_hbm, out_hbm, idx_v, rows_v, *sems):
        wid = lax.axis_index("s") * _NC + lax.axis_index("c")
        base = wid * rows_per_w
        gsems = list(sems[:nbuf])
        osems = list(sems[nbuf:])

        pltpu.sync_copy(idx_hbm.at[wid], idx_v)

        for b in range(nbuf):
            pltpu.async_copy(g_hbm.at[idx_v.at[b]], rows_v.at[b], gsems[b])

        @pl.loop(0, n_super)
        def _super(g):
            for b in range(nbuf):
                c = g * nbuf + b
                pltpu.make_async_copy(
                    g_hbm.at[pl.ds(0, chunk)], rows_v.at[b], gsems[b]).wait()
                pltpu.async_copy(
                    rows_v.at[b],
                    out_hbm.at[pl.ds(base + c * chunk, chunk)], osems[b])
                pltpu.make_async_copy(
                    rows_v.at[b], out_hbm.at[pl.ds(0, chunk)],
                    osems[b]).wait()

                @pl.when(g < n_super - 1)
                def _():
                    pltpu.async_copy(g_hbm.at[idx_v.at[c + nbuf]],
                                     rows_v.at[b], gsems[b])

    return gather_k


def _mlp_body(vf_ref, img_ref, w1a_ref, b1_ref, w2_ref, b2_ref, out_ref):
    h = jnp.dot(vf_ref[...], w1a_ref[...], preferred_element_type=jnp.float32)
    h = jnp.maximum(h + img_ref[...] + b1_ref[...], 0.0)
    out_ref[...] = (jnp.dot(h, w2_ref[...], preferred_element_type=jnp.float32)
                    + b2_ref[...])


def _mlp(vf, img, w1a, b1, w2, b2, tile):
    n, pfd = vf.shape
    hidden = w1a.shape[1]
    out_dim = w2.shape[1]
    return pl.pallas_call(
        _mlp_body,
        grid=(n // tile,),
        in_specs=[
            pl.BlockSpec((tile, pfd), lambda i: (i, 0)),
            pl.BlockSpec((tile, hidden), lambda i: (i, 0)),
            pl.BlockSpec((pfd, hidden), lambda i: (0, 0)),
            pl.BlockSpec((1, hidden), lambda i: (0, 0)),
            pl.BlockSpec((hidden, out_dim), lambda i: (0, 0)),
            pl.BlockSpec((1, out_dim), lambda i: (0, 0)),
        ],
        out_specs=pl.BlockSpec((tile, out_dim), lambda i: (i, 0)),
        out_shape=jax.ShapeDtypeStruct((n, out_dim), jnp.float32),
    )(vf, img, w1a, b1, w2, b2)


def kernel(patch_tokens, voxel_features, voxel_coords, image_sizes, K, Rt,
           W1, b1, W2, b2):
    B, nv, M, dim = patch_tokens.shape
    V = voxel_features.shape[1]
    pfd = voxel_features.shape[2]
    hidden = W1.shape[1]
    out_dim = W2.shape[1]
    n_rows = B * V

    def _bf(a):
        return a.astype(jnp.bfloat16).astype(jnp.float32)

    xs = _bf(voxel_coords[..., 0][:, None, :])
    ys = _bf(voxel_coords[..., 1][:, None, :])
    zs = _bf(voxel_coords[..., 2][:, None, :])
    params = _bf(jnp.concatenate([Rt.reshape(B, 1, 12), K.reshape(B, 1, 9)],
                                 axis=2))
    scale = (_RESIZE / image_sizes[0].astype(jnp.float32)).reshape(1, 2)

    idx3, g = _prep(params, scale, xs, ys, zs, patch_tokens, W1[pfd:, :])

    chunk, nbuf = 64, 4
    n_chunks = n_rows // _NW // chunk
    gather_k = _make_sc_gather(n_rows, B * M, hidden, chunk=chunk, nbuf=nbuf)
    img = gather_k(idx3.reshape(_NW, n_chunks, chunk), g.reshape(B * M, hidden))

    out = _mlp(voxel_features.reshape(n_rows, pfd), img, W1[:pfd, :],
               b1.reshape(1, hidden), W2, b2.reshape(1, out_dim), tile=2048)
    return out.reshape(B, V, out_dim)
